# Initial kernel scaffold; baseline (speedup 1.0000x reference)
import jax
import jax.numpy as jnp
from jax.experimental import pallas as pl


def kernel(users, movies, user_table, movie_table, W1, b1, W2, b2, W3, b3):
    raise NotImplementedError("write your pallas kernel here")


# speedup vs baseline: 2.8659x; 2.8659x over previous
import functools

import jax
import jax.numpy as jnp
from jax import lax
from jax.experimental import pallas as pl
from jax.experimental.pallas import tpu as pltpu
from jax.experimental.pallas import tpu_sc as plsc

NC = 2
NS = 16
NW = NC * NS
CHUNK = 128


def _gather_body(uidx, midx, utab, mtab, uout, mout, idx_v, rows_v, sem):
    nch = idx_v.shape[0]
    bpw = nch * CHUNK
    wid = lax.axis_index("s") * NC + lax.axis_index("c")
    base = wid * bpw

    def one_table(idx_hbm, tab_hbm, out_hbm):
        pltpu.sync_copy(idx_hbm.at[wid], idx_v)
        copies = []
        for j in range(nch):
            copies.append(
                pltpu.async_copy(
                    tab_hbm.at[idx_v.at[j]],
                    rows_v.at[pl.ds(j * CHUNK, CHUNK)],
                    sem,
                )
            )
        for c in copies:
            c.wait()
        pltpu.sync_copy(rows_v, out_hbm.at[pl.ds(base, bpw)])

    one_table(uidx, utab, uout)
    one_table(midx, mtab, mout)


def _mlp_body(xu_ref, xm_ref, w1a_ref, w1b_ref, b1_ref, w2_ref, b2_ref,
              w3_ref, b3_ref, out_ref):
    h = (jnp.dot(xu_ref[...], w1a_ref[...], preferred_element_type=jnp.float32)
         + jnp.dot(xm_ref[...], w1b_ref[...], preferred_element_type=jnp.float32)
         + b1_ref[...])
    h = jnp.maximum(h, 0.0)
    h = jnp.maximum(
        jnp.dot(h, w2_ref[...], preferred_element_type=jnp.float32) + b2_ref[...],
        0.0)
    o = jnp.maximum(
        jnp.dot(h, w3_ref[...], preferred_element_type=jnp.float32) + b3_ref[...],
        0.0)
    out_ref[...] = o


def kernel(users, movies, user_table, movie_table, W1, b1, W2, b2, W3, b3):
    B = users.shape[0]
    D = user_table.shape[1]
    bpw = B // NW
    nch = bpw // CHUNK

    uidx = users.astype(jnp.int32).reshape(NW, nch, CHUNK)
    midx = movies.astype(jnp.int32).reshape(NW, nch, CHUNK)

    mesh = plsc.VectorSubcoreMesh(core_axis_name="c", subcore_axis_name="s")
    gather = pl.kernel(
        _gather_body,
        out_type=[
            jax.ShapeDtypeStruct((B, D), jnp.float32),
            jax.ShapeDtypeStruct((B, D), jnp.float32),
        ],
        mesh=mesh,
        scratch_types=[
            pltpu.VMEM((nch, CHUNK), jnp.int32),
            pltpu.VMEM((bpw, D), jnp.float32),
            pltpu.SemaphoreType.DMA,
        ],
    )
    ue, me = gather(uidx, midx, user_table, movie_table)

    BLK = 2048
    grid = (B // BLK,)
    mlp = pl.pallas_call(
        _mlp_body,
        grid=grid,
        in_specs=[
            pl.BlockSpec((BLK, D), lambda i: (i, 0)),
            pl.BlockSpec((BLK, D), lambda i: (i, 0)),
            pl.BlockSpec((D, 64), lambda i: (0, 0)),
            pl.BlockSpec((D, 64), lambda i: (0, 0)),
            pl.BlockSpec((1, 64), lambda i: (0, 0)),
            pl.BlockSpec((64, 16), lambda i: (0, 0)),
            pl.BlockSpec((1, 16), lambda i: (0, 0)),
            pl.BlockSpec((16, 1), lambda i: (0, 0)),
            pl.BlockSpec((1, 1), lambda i: (0, 0)),
        ],
        out_specs=pl.BlockSpec((BLK, 1), lambda i: (i, 0)),
        out_shape=jax.ShapeDtypeStruct((B, 1), jnp.float32),
    )
    out = mlp(ue, me, W1[:D], W1[D:], b1.reshape(1, -1), W2,
              b2.reshape(1, -1), W3, b3.reshape(1, -1))
    return out.reshape(B)


# speedup vs baseline: 2.9013x; 1.0124x over previous
import functools

import jax
import jax.numpy as jnp
from jax import lax
from jax.experimental import pallas as pl
from jax.experimental.pallas import tpu as pltpu
from jax.experimental.pallas import tpu_sc as plsc

NC = 2
NS = 16
NW = NC * NS
CHUNK = 128


def _gather_body(uidx, midx, utab, mtab, uout, mout, idx_v, rows_v, gsem, wsem):
    bpw = idx_v.shape[1]
    nch = bpw // CHUNK
    depth = rows_v.shape[0]
    wid = lax.axis_index("s") * NC + lax.axis_index("c")
    base = wid * bpw

    pltpu.sync_copy(uidx.at[pl.ds(base, bpw)], idx_v.at[0])
    pltpu.sync_copy(midx.at[pl.ds(base, bpw)], idx_v.at[1])

    tasks = [(t, j, tab, out)
             for t, (tab, out) in enumerate(((utab, uout), (mtab, mout)))
             for j in range(nch)]
    n = len(tasks)

    def fire_gather(k):
        t, j, tab, _ = tasks[k]
        return pltpu.async_copy(
            tab.at[idx_v.at[t, pl.ds(j * CHUNK, CHUNK)]],
            rows_v.at[k % depth], gsem.at[k % depth])

    gathers = [None] * n
    writes = [None] * n
    for k in range(min(depth, n)):
        gathers[k] = fire_gather(k)
    for k in range(n):
        t, j, _, out = tasks[k]
        gathers[k].wait()
        writes[k] = pltpu.async_copy(
            rows_v.at[k % depth],
            out.at[pl.ds(base + j * CHUNK, CHUNK)], wsem.at[k % depth])
        kn = k + depth
        if kn < n:
            writes[k].wait()
            gathers[kn] = fire_gather(kn)
    for k in range(max(0, n - depth), n):
        writes[k].wait()


def _mlp_body(xu_ref, xm_ref, w1a_ref, w1b_ref, b1_ref, w2_ref, b2_ref,
              w3_ref, b3_ref, out_ref):
    h = (jnp.dot(xu_ref[...], w1a_ref[...], preferred_element_type=jnp.float32)
         + jnp.dot(xm_ref[...], w1b_ref[...], preferred_element_type=jnp.float32)
         + b1_ref[...])
    h = jnp.maximum(h, 0.0)
    h = jnp.maximum(
        jnp.dot(h, w2_ref[...], preferred_element_type=jnp.float32) + b2_ref[...],
        0.0)
    o = jnp.maximum(
        jnp.dot(h, w3_ref[...], preferred_element_type=jnp.float32) + b3_ref[...],
        0.0)
    out_ref[...] = o


def kernel(users, movies, user_table, movie_table, W1, b1, W2, b2, W3, b3):
    B = users.shape[0]
    D = user_table.shape[1]
    bpw = B // NW
    depth = 7

    uidx = users.astype(jnp.int32)
    midx = movies.astype(jnp.int32)

    mesh = plsc.VectorSubcoreMesh(core_axis_name="c", subcore_axis_name="s")
    gather = pl.kernel(
        _gather_body,
        out_type=[
            jax.ShapeDtypeStruct((B, D), jnp.float32),
            jax.ShapeDtypeStruct((B, D), jnp.float32),
        ],
        mesh=mesh,
        scratch_types=[
            pltpu.VMEM((2, bpw), jnp.int32),
            pltpu.VMEM((depth, CHUNK, D), jnp.float32),
            pltpu.SemaphoreType.DMA((depth,)),
            pltpu.SemaphoreType.DMA((depth,)),
        ],
    )
    ue, me = gather(uidx, midx, user_table, movie_table)

    BLK = 2048
    grid = (B // BLK,)
    mlp = pl.pallas_call(
        _mlp_body,
        grid=grid,
        in_specs=[
            pl.BlockSpec((BLK, D), lambda i: (i, 0)),
            pl.BlockSpec((BLK, D), lambda i: (i, 0)),
            pl.BlockSpec((D, 64), lambda i: (0, 0)),
            pl.BlockSpec((D, 64), lambda i: (0, 0)),
            pl.BlockSpec((1, 64), lambda i: (0, 0)),
            pl.BlockSpec((64, 16), lambda i: (0, 0)),
            pl.BlockSpec((1, 16), lambda i: (0, 0)),
            pl.BlockSpec((16, 1), lambda i: (0, 0)),
            pl.BlockSpec((1, 1), lambda i: (0, 0)),
        ],
        out_specs=pl.BlockSpec((BLK, 1), lambda i: (i, 0)),
        out_shape=jax.ShapeDtypeStruct((B, 1), jnp.float32),
    )
    out = mlp(ue, me, W1[:D], W1[D:], b1.reshape(1, -1), W2,
              b2.reshape(1, -1), W3, b3.reshape(1, -1))
    return out.reshape(B)
